# baseline (speedup 1.0000x reference)
import jax
import jax.numpy as jnp
from jax import lax
from jax.experimental import pallas as pl
from jax.experimental.pallas import tpu as pltpu
from jax.experimental.pallas import tpu_sc as plsc

NUM_ROUTED_E = 8
NUM_GROUPS = 9
HIDDEN_D = 768
FFN_D = 1024
TOKENS_N = 2048
BLK = 256
NBLK = TOKENS_N // BLK + NUM_GROUPS + 1
NP = NBLK * BLK
ROWS_R = 16
COLS_C = 128
CHUNK = 64
HALF = CHUNK // 2
WSW = 128


def _route_body(eid_ref, pos_ref, bexp_ref):
    e2 = eid_ref[...]
    col = lax.broadcasted_iota(jnp.int32, (COLS_C, COLS_C), 0)
    row = lax.broadcasted_iota(jnp.int32, (COLS_C, COLS_C), 1)
    upper_c = jnp.where(col < row, 1.0, 0.0)
    colr = lax.broadcasted_iota(jnp.int32, (ROWS_R, ROWS_R), 0)
    rowr = lax.broadcasted_iota(jnp.int32, (ROWS_R, ROWS_R), 1)
    lower_r = jnp.where(rowr < colr, 1.0, 0.0)

    pos = jnp.zeros((ROWS_R, COLS_C), jnp.float32)
    pstart = jnp.float32(0.0)
    pstarts = []
    for g in range(NUM_GROUPS):
        if g == 0:
            m = jnp.where(e2 >= NUM_ROUTED_E, 1.0, 0.0)
        else:
            m = jnp.where(e2 == g - 1, 1.0, 0.0)
        wpref = lax.dot_general(m, upper_c, (((1,), (0,)), ((), ())),
                                preferred_element_type=jnp.float32)
        s = jnp.sum(m, axis=1, keepdims=True)
        rp = lax.dot_general(lower_r, s, (((1,), (0,)), ((), ())),
                             preferred_element_type=jnp.float32)
        rank = rp + wpref
        pstarts.append(pstart)
        pos = pos + m * (pstart + rank)
        cnt_i = jnp.sum(s).astype(jnp.int32)
        pcnt = ((cnt_i + BLK - 1) & ~(BLK - 1)).astype(jnp.float32)
        pstart = pstart + pcnt

    pos_ref[...] = pos.astype(jnp.int32)

    bv = (lax.broadcasted_iota(jnp.int32, (1, COLS_C), 1) * BLK
          ).astype(jnp.float32)
    ge = jnp.zeros((1, COLS_C), jnp.float32)
    for g in range(NUM_GROUPS):
        ge = ge + jnp.where(bv >= pstarts[g], 1.0, 0.0)
    ge = ge + jnp.where(bv >= pstart, 1.0, 0.0)
    bexp_ref[...] = (ge - 1.0).astype(jnp.int32)


def _sc_scatter_body(x_hbm, wt16_hbm, pos_hbm, xs_hbm, ws_hbm,
                     idx0, idx1, rows0, rows1, wrow0, wrow1,
                     sem_a, sem_b, sem_w):
    wid = lax.axis_index("s") * 2 + lax.axis_index("c")
    base = wid * CHUNK
    cx0 = pltpu.async_copy(x_hbm.at[pl.ds(base, HALF)], rows0, sem_a)
    cx1 = pltpu.async_copy(x_hbm.at[pl.ds(base + HALF, HALF)], rows1, sem_b)
    cw0 = pltpu.async_copy(wt16_hbm.at[pl.ds(base, HALF)], wrow0, sem_w)
    cw1 = pltpu.async_copy(wt16_hbm.at[pl.ds(base + HALF, HALF)], wrow1,
                           sem_w)
    pltpu.sync_copy(pos_hbm.at[pl.ds(base, HALF)], idx0)
    pltpu.sync_copy(pos_hbm.at[pl.ds(base + HALF, HALF)], idx1)
    cx0.wait()
    sx0 = pltpu.async_copy(rows0, xs_hbm.at[idx0], sem_a)
    cx1.wait()
    sx1 = pltpu.async_copy(rows1, xs_hbm.at[idx1], sem_b)
    cw0.wait()
    cw1.wait()
    sw0 = pltpu.async_copy(wrow0, ws_hbm.at[idx0], sem_w)
    sw1 = pltpu.async_copy(wrow1, ws_hbm.at[idx1], sem_w)
    sx0.wait()
    sx1.wait()
    sw0.wait()
    sw1.wait()


def _sc_gather_body(ys_hbm, pos_hbm, g_hbm, idx0, idx1, rows0, rows1,
                    sem_a, sem_b):
    wid = lax.axis_index("s") * 2 + lax.axis_index("c")
    base = wid * CHUNK
    pltpu.sync_copy(pos_hbm.at[pl.ds(base, HALF)], idx0)
    g0 = pltpu.async_copy(ys_hbm.at[idx0], rows0, sem_a)
    pltpu.sync_copy(pos_hbm.at[pl.ds(base + HALF, HALF)], idx1)
    g1 = pltpu.async_copy(ys_hbm.at[idx1], rows1, sem_b)
    g0.wait()
    s0 = pltpu.async_copy(rows0, g_hbm.at[pl.ds(base, HALF)], sem_a)
    g1.wait()
    s1 = pltpu.async_copy(rows1, g_hbm.at[pl.ds(base + HALF, HALF)], sem_b)
    s0.wait()
    s1.wait()


def _ffn_body(bexp_ref, xs_ref, ws_ref, gupg_ref, gupu_ref, dwn_ref, ys_ref):
    e = bexp_ref[pl.program_id(0)]
    ws1 = ws_ref[:, :1]

    @pl.when(jnp.logical_and(e >= 1, e <= NUM_ROUTED_E))
    def _routed():
        g = lax.dot_general(xs_ref[...], gupg_ref[0],
                            (((1,), (1,)), ((), ())),
                            preferred_element_type=jnp.float32)
        u = lax.dot_general(xs_ref[...], gupu_ref[0],
                            (((1,), (1,)), ((), ())),
                            preferred_element_type=jnp.float32)
        h = g * jax.nn.sigmoid(g) * u
        y = lax.dot_general(h, dwn_ref[0], (((1,), (1,)), ((), ())),
                            preferred_element_type=jnp.float32)
        ys_ref[...] = ws1 * y

    @pl.when(jnp.logical_or(e == 0, e > NUM_ROUTED_E))
    def _identity():
        ys_ref[...] = ws1 * xs_ref[...]


def _ffn_call(bexp, xs, ws, gate_up_proj, down_proj):
    def _blk(b, be):
        return jnp.where(be[b] > NUM_ROUTED_E, NBLK - 1, b)

    def _exp(b, be):
        return jnp.clip(be[b] - 1, 0, 7)

    return pl.pallas_call(
        _ffn_body,
        grid_spec=pltpu.PrefetchScalarGridSpec(
            num_scalar_prefetch=1,
            grid=(NBLK,),
            in_specs=[
                pl.BlockSpec((BLK, HIDDEN_D), lambda b, be: (_blk(b, be), 0)),
                pl.BlockSpec((BLK, WSW), lambda b, be: (_blk(b, be), 0)),
                pl.BlockSpec((1, FFN_D, HIDDEN_D),
                             lambda b, be: (_exp(b, be), 0, 0)),
                pl.BlockSpec((1, FFN_D, HIDDEN_D),
                             lambda b, be: (_exp(b, be), 1, 0)),
                pl.BlockSpec((1, HIDDEN_D, FFN_D),
                             lambda b, be: (_exp(b, be), 0, 0)),
            ],
            out_specs=pl.BlockSpec((BLK, HIDDEN_D),
                                   lambda b, be: (_blk(b, be), 0)),
        ),
        out_shape=jax.ShapeDtypeStruct((NP, HIDDEN_D), jnp.float32),
        compiler_params=pltpu.CompilerParams(
            dimension_semantics=("arbitrary",),
        ),
    )(bexp, xs, ws, gate_up_proj, gate_up_proj, down_proj)


def kernel(hidden_states, top_k_index, top_k_weights, gate_up_proj, down_proj):
    T, H = hidden_states.shape
    e2 = top_k_index.reshape(ROWS_R, COLS_C)

    pos2, bexp2 = pl.pallas_call(
        _route_body,
        out_shape=(
            jax.ShapeDtypeStruct((ROWS_R, COLS_C), jnp.int32),
            jax.ShapeDtypeStruct((1, COLS_C), jnp.int32),
        ),
    )(e2)
    pos = pos2.reshape(T)
    bexp = bexp2.reshape(COLS_C)

    scmesh = plsc.VectorSubcoreMesh(core_axis_name="c", subcore_axis_name="s")

    wt16 = jnp.tile(top_k_weights, (1, WSW))

    xs, ws = pl.kernel(
        _sc_scatter_body,
        out_type=(
            jax.ShapeDtypeStruct((NP, H), jnp.float32),
            jax.ShapeDtypeStruct((NP, WSW), jnp.float32),
        ),
        mesh=scmesh,
        scratch_types=[
            pltpu.VMEM((HALF,), jnp.int32),
            pltpu.VMEM((HALF,), jnp.int32),
            pltpu.VMEM((HALF, H), jnp.float32),
            pltpu.VMEM((HALF, H), jnp.float32),
            pltpu.VMEM((HALF, WSW), jnp.float32),
            pltpu.VMEM((HALF, WSW), jnp.float32),
            pltpu.SemaphoreType.DMA,
            pltpu.SemaphoreType.DMA,
            pltpu.SemaphoreType.DMA,
        ],
    )(hidden_states, wt16, pos)

    ys = _ffn_call(bexp, xs, ws, gate_up_proj, down_proj)

    out = pl.kernel(
        _sc_gather_body,
        out_type=jax.ShapeDtypeStruct((T, H), jnp.float32),
        mesh=scmesh,
        scratch_types=[
            pltpu.VMEM((HALF,), jnp.int32),
            pltpu.VMEM((HALF,), jnp.int32),
            pltpu.VMEM((HALF, H), jnp.float32),
            pltpu.VMEM((HALF, H), jnp.float32),
            pltpu.SemaphoreType.DMA,
            pltpu.SemaphoreType.DMA,
        ],
    )(ys, pos)
    return out


# speedup vs baseline: 1.0089x; 1.0089x over previous
import jax
import jax.numpy as jnp
from jax import lax
from jax.experimental import pallas as pl
from jax.experimental.pallas import tpu as pltpu
from jax.experimental.pallas import tpu_sc as plsc

NUM_ROUTED_E = 8
NUM_GROUPS = 9
HIDDEN_D = 768
FFN_D = 1024
TOKENS_N = 2048
BLK = 256
NBLK = TOKENS_N // BLK + NUM_GROUPS + 1
NP = NBLK * BLK
ROWS_R = 16
COLS_C = 128
CHUNK = 64
HALF = CHUNK // 2
WSW = 128


def _route_body(eid_ref, pos_ref, bexp_ref):
    e2 = eid_ref[...]
    col = lax.broadcasted_iota(jnp.int32, (COLS_C, COLS_C), 0)
    row = lax.broadcasted_iota(jnp.int32, (COLS_C, COLS_C), 1)
    upper_c = jnp.where(col < row, 1.0, 0.0)
    colr = lax.broadcasted_iota(jnp.int32, (ROWS_R, ROWS_R), 0)
    rowr = lax.broadcasted_iota(jnp.int32, (ROWS_R, ROWS_R), 1)
    lower_r = jnp.where(rowr < colr, 1.0, 0.0)

    pos = jnp.zeros((ROWS_R, COLS_C), jnp.float32)
    pstart = jnp.float32(0.0)
    pstarts = []
    for g in range(NUM_GROUPS):
        if g < NUM_ROUTED_E:
            m = jnp.where(e2 == g, 1.0, 0.0)
        else:
            m = jnp.where(e2 >= NUM_ROUTED_E, 1.0, 0.0)
        wpref = lax.dot_general(m, upper_c, (((1,), (0,)), ((), ())),
                                preferred_element_type=jnp.float32)
        s = jnp.sum(m, axis=1, keepdims=True)
        rp = lax.dot_general(lower_r, s, (((1,), (0,)), ((), ())),
                             preferred_element_type=jnp.float32)
        rank = rp + wpref
        pstarts.append(pstart)
        pos = pos + m * (pstart + rank)
        cnt_i = jnp.sum(s).astype(jnp.int32)
        pcnt = ((cnt_i + BLK - 1) & ~(BLK - 1)).astype(jnp.float32)
        pstart = pstart + pcnt

    pos_ref[...] = pos.astype(jnp.int32)

    bv = (lax.broadcasted_iota(jnp.int32, (1, COLS_C), 1) * BLK
          ).astype(jnp.float32)
    ge = jnp.zeros((1, COLS_C), jnp.float32)
    for g in range(NUM_GROUPS):
        ge = ge + jnp.where(bv >= pstarts[g], 1.0, 0.0)
    ge = ge + jnp.where(bv >= pstart, 1.0, 0.0)
    bexp_ref[...] = (ge - 1.0).astype(jnp.int32)


def _sc_scatter_body(x_hbm, wt16_hbm, pos_hbm, xs_hbm, ws_hbm,
                     idx0, idx1, rows0, rows1, wrow0, wrow1,
                     sem_a, sem_b, sem_w):
    wid = lax.axis_index("s") * 2 + lax.axis_index("c")
    base = wid * CHUNK
    cx0 = pltpu.async_copy(x_hbm.at[pl.ds(base, HALF)], rows0, sem_a)
    cx1 = pltpu.async_copy(x_hbm.at[pl.ds(base + HALF, HALF)], rows1, sem_b)
    cw0 = pltpu.async_copy(wt16_hbm.at[pl.ds(base, HALF)], wrow0, sem_w)
    cw1 = pltpu.async_copy(wt16_hbm.at[pl.ds(base + HALF, HALF)], wrow1,
                           sem_w)
    pltpu.sync_copy(pos_hbm.at[pl.ds(base, HALF)], idx0)
    pltpu.sync_copy(pos_hbm.at[pl.ds(base + HALF, HALF)], idx1)
    cx0.wait()
    sx0 = pltpu.async_copy(rows0, xs_hbm.at[idx0], sem_a)
    cx1.wait()
    sx1 = pltpu.async_copy(rows1, xs_hbm.at[idx1], sem_b)
    cw0.wait()
    cw1.wait()
    sw0 = pltpu.async_copy(wrow0, ws_hbm.at[idx0], sem_w)
    sw1 = pltpu.async_copy(wrow1, ws_hbm.at[idx1], sem_w)
    sx0.wait()
    sx1.wait()
    sw0.wait()
    sw1.wait()


def _sc_gather_body(ys_hbm, pos_hbm, g_hbm, idx0, idx1, rows0, rows1,
                    sem_a, sem_b):
    wid = lax.axis_index("s") * 2 + lax.axis_index("c")
    base = wid * CHUNK
    pltpu.sync_copy(pos_hbm.at[pl.ds(base, HALF)], idx0)
    g0 = pltpu.async_copy(ys_hbm.at[idx0], rows0, sem_a)
    pltpu.sync_copy(pos_hbm.at[pl.ds(base + HALF, HALF)], idx1)
    g1 = pltpu.async_copy(ys_hbm.at[idx1], rows1, sem_b)
    g0.wait()
    s0 = pltpu.async_copy(rows0, g_hbm.at[pl.ds(base, HALF)], sem_a)
    g1.wait()
    s1 = pltpu.async_copy(rows1, g_hbm.at[pl.ds(base + HALF, HALF)], sem_b)
    s0.wait()
    s1.wait()


def _ffn_body(bexp_ref, xs_ref, ws_ref, gupg_ref, gupu_ref, dwn_ref, ys_ref):
    e = bexp_ref[pl.program_id(0)]
    ws1 = ws_ref[:, :1]

    @pl.when(e < NUM_ROUTED_E)
    def _routed():
        g = lax.dot_general(xs_ref[...], gupg_ref[0],
                            (((1,), (1,)), ((), ())),
                            preferred_element_type=jnp.float32)
        u = lax.dot_general(xs_ref[...], gupu_ref[0],
                            (((1,), (1,)), ((), ())),
                            preferred_element_type=jnp.float32)
        h = g * jax.nn.sigmoid(g) * u
        y = lax.dot_general(h, dwn_ref[0], (((1,), (1,)), ((), ())),
                            preferred_element_type=jnp.float32)
        ys_ref[...] = ws1 * y

    @pl.when(e >= NUM_ROUTED_E)
    def _identity():
        ys_ref[...] = ws1 * xs_ref[...]


def _ffn_call(bexp, xs, ws, gate_up_proj, down_proj):
    def _blk(b, be):
        return jnp.where(be[b] > NUM_ROUTED_E, NBLK - 1, b)

    def _exp(b, be):
        return jnp.minimum(be[b], 7)

    return pl.pallas_call(
        _ffn_body,
        grid_spec=pltpu.PrefetchScalarGridSpec(
            num_scalar_prefetch=1,
            grid=(NBLK,),
            in_specs=[
                pl.BlockSpec((BLK, HIDDEN_D), lambda b, be: (_blk(b, be), 0)),
                pl.BlockSpec((BLK, WSW), lambda b, be: (_blk(b, be), 0)),
                pl.BlockSpec((1, FFN_D, HIDDEN_D),
                             lambda b, be: (_exp(b, be), 0, 0)),
                pl.BlockSpec((1, FFN_D, HIDDEN_D),
                             lambda b, be: (_exp(b, be), 1, 0)),
                pl.BlockSpec((1, HIDDEN_D, FFN_D),
                             lambda b, be: (_exp(b, be), 0, 0)),
            ],
            out_specs=pl.BlockSpec((BLK, HIDDEN_D),
                                   lambda b, be: (_blk(b, be), 0)),
        ),
        out_shape=jax.ShapeDtypeStruct((NP, HIDDEN_D), jnp.float32),
        compiler_params=pltpu.CompilerParams(
            dimension_semantics=("arbitrary",),
        ),
    )(bexp, xs, ws, gate_up_proj, gate_up_proj, down_proj)


def kernel(hidden_states, top_k_index, top_k_weights, gate_up_proj, down_proj):
    T, H = hidden_states.shape
    e2 = top_k_index.reshape(ROWS_R, COLS_C)

    pos2, bexp2 = pl.pallas_call(
        _route_body,
        out_shape=(
            jax.ShapeDtypeStruct((ROWS_R, COLS_C), jnp.int32),
            jax.ShapeDtypeStruct((1, COLS_C), jnp.int32),
        ),
    )(e2)
    pos = pos2.reshape(T)
    bexp = bexp2.reshape(COLS_C)

    scmesh = plsc.VectorSubcoreMesh(core_axis_name="c", subcore_axis_name="s")

    wt16 = jnp.tile(top_k_weights, (1, WSW))

    xs, ws = pl.kernel(
        _sc_scatter_body,
        out_type=(
            jax.ShapeDtypeStruct((NP, H), jnp.float32),
            jax.ShapeDtypeStruct((NP, WSW), jnp.float32),
        ),
        mesh=scmesh,
        scratch_types=[
            pltpu.VMEM((HALF,), jnp.int32),
            pltpu.VMEM((HALF,), jnp.int32),
            pltpu.VMEM((HALF, H), jnp.float32),
            pltpu.VMEM((HALF, H), jnp.float32),
            pltpu.VMEM((HALF, WSW), jnp.float32),
            pltpu.VMEM((HALF, WSW), jnp.float32),
            pltpu.SemaphoreType.DMA,
            pltpu.SemaphoreType.DMA,
            pltpu.SemaphoreType.DMA,
        ],
    )(hidden_states, wt16, pos)

    ys = _ffn_call(bexp, xs, ws, gate_up_proj, down_proj)

    out = pl.kernel(
        _sc_gather_body,
        out_type=jax.ShapeDtypeStruct((T, H), jnp.float32),
        mesh=scmesh,
        scratch_types=[
            pltpu.VMEM((HALF,), jnp.int32),
            pltpu.VMEM((HALF,), jnp.int32),
            pltpu.VMEM((HALF, H), jnp.float32),
            pltpu.VMEM((HALF, H), jnp.float32),
            pltpu.SemaphoreType.DMA,
            pltpu.SemaphoreType.DMA,
        ],
    )(ys, pos)
    return out


# speedup vs baseline: 1.0166x; 1.0076x over previous
import jax
import jax.numpy as jnp
from jax import lax
from jax.experimental import pallas as pl
from jax.experimental.pallas import tpu as pltpu
from jax.experimental.pallas import tpu_sc as plsc

NUM_ROUTED_E = 8
NUM_GROUPS = 9
HIDDEN_D = 768
FFN_D = 1024
TOKENS_N = 2048
BLK = 256
NBLK = TOKENS_N // BLK + NUM_GROUPS - 1
NP = NBLK * BLK
ROWS_R = 16
COLS_C = 128
CHUNK = 64
HALF = CHUNK // 2
WSW = 128


def _route_body(eid_ref, pos_ref, bexp_ref):
    e2 = eid_ref[...]
    col = lax.broadcasted_iota(jnp.int32, (COLS_C, COLS_C), 0)
    row = lax.broadcasted_iota(jnp.int32, (COLS_C, COLS_C), 1)
    upper_c = jnp.where(col < row, 1.0, 0.0)
    colr = lax.broadcasted_iota(jnp.int32, (ROWS_R, ROWS_R), 0)
    rowr = lax.broadcasted_iota(jnp.int32, (ROWS_R, ROWS_R), 1)
    lower_r = jnp.where(rowr < colr, 1.0, 0.0)

    pos = jnp.zeros((ROWS_R, COLS_C), jnp.float32)
    pstart = jnp.float32(0.0)
    pstarts = []
    for g in range(NUM_GROUPS):
        if g < NUM_ROUTED_E:
            m = jnp.where(e2 == g, 1.0, 0.0)
        else:
            m = jnp.where(e2 >= NUM_ROUTED_E, 1.0, 0.0)
        wpref = lax.dot_general(m, upper_c, (((1,), (0,)), ((), ())),
                                preferred_element_type=jnp.float32)
        s = jnp.sum(m, axis=1, keepdims=True)
        rp = lax.dot_general(lower_r, s, (((1,), (0,)), ((), ())),
                             preferred_element_type=jnp.float32)
        rank = rp + wpref
        pstarts.append(pstart)
        pos = pos + m * (pstart + rank)
        cnt_i = jnp.sum(s).astype(jnp.int32)
        pcnt = ((cnt_i + BLK - 1) & ~(BLK - 1)).astype(jnp.float32)
        pstart = pstart + pcnt

    pos_ref[...] = pos.astype(jnp.int32)

    bv = (lax.broadcasted_iota(jnp.int32, (1, COLS_C), 1) * BLK
          ).astype(jnp.float32)
    ge = jnp.zeros((1, COLS_C), jnp.float32)
    for g in range(NUM_GROUPS):
        ge = ge + jnp.where(bv >= pstarts[g], 1.0, 0.0)
    ge = ge + jnp.where(bv >= pstart, 1.0, 0.0)
    bexp_ref[...] = (ge - 1.0).astype(jnp.int32)


def _sc_scatter_body(x_hbm, wt16_hbm, pos_hbm, xs_hbm, ws_hbm,
                     idx0, idx1, rows0, rows1, wrow0, wrow1,
                     sem_a, sem_b, sem_w):
    wid = lax.axis_index("s") * 2 + lax.axis_index("c")
    base = wid * CHUNK
    cx0 = pltpu.async_copy(x_hbm.at[pl.ds(base, HALF)], rows0, sem_a)
    cx1 = pltpu.async_copy(x_hbm.at[pl.ds(base + HALF, HALF)], rows1, sem_b)
    cw0 = pltpu.async_copy(wt16_hbm.at[pl.ds(base, HALF)], wrow0, sem_w)
    cw1 = pltpu.async_copy(wt16_hbm.at[pl.ds(base + HALF, HALF)], wrow1,
                           sem_w)
    pltpu.sync_copy(pos_hbm.at[pl.ds(base, HALF)], idx0)
    pltpu.sync_copy(pos_hbm.at[pl.ds(base + HALF, HALF)], idx1)
    cx0.wait()
    sx0 = pltpu.async_copy(rows0, xs_hbm.at[idx0], sem_a)
    cx1.wait()
    sx1 = pltpu.async_copy(rows1, xs_hbm.at[idx1], sem_b)
    cw0.wait()
    cw1.wait()
    sw0 = pltpu.async_copy(wrow0, ws_hbm.at[idx0], sem_w)
    sw1 = pltpu.async_copy(wrow1, ws_hbm.at[idx1], sem_w)
    sx0.wait()
    sx1.wait()
    sw0.wait()
    sw1.wait()


def _sc_gather_body(ys_hbm, pos_hbm, g_hbm, idx0, idx1, rows0, rows1,
                    sem_a, sem_b):
    wid = lax.axis_index("s") * 2 + lax.axis_index("c")
    base = wid * CHUNK
    pltpu.sync_copy(pos_hbm.at[pl.ds(base, HALF)], idx0)
    g0 = pltpu.async_copy(ys_hbm.at[idx0], rows0, sem_a)
    pltpu.sync_copy(pos_hbm.at[pl.ds(base + HALF, HALF)], idx1)
    g1 = pltpu.async_copy(ys_hbm.at[idx1], rows1, sem_b)
    g0.wait()
    s0 = pltpu.async_copy(rows0, g_hbm.at[pl.ds(base, HALF)], sem_a)
    g1.wait()
    s1 = pltpu.async_copy(rows1, g_hbm.at[pl.ds(base + HALF, HALF)], sem_b)
    s0.wait()
    s1.wait()


def _ffn_body(bexp_ref, xs_ref, ws_ref, gupg_ref, gupu_ref, dwn_ref, ys_ref):
    e = bexp_ref[pl.program_id(0)]
    ws1 = ws_ref[:, :1]

    @pl.when(e < NUM_ROUTED_E)
    def _routed():
        g = lax.dot_general(xs_ref[...], gupg_ref[0],
                            (((1,), (1,)), ((), ())),
                            preferred_element_type=jnp.float32)
        u = lax.dot_general(xs_ref[...], gupu_ref[0],
                            (((1,), (1,)), ((), ())),
                            preferred_element_type=jnp.float32)
        h = g * jax.nn.sigmoid(g) * u
        y = lax.dot_general(h, dwn_ref[0], (((1,), (1,)), ((), ())),
                            preferred_element_type=jnp.float32)
        ys_ref[...] = ws1 * y

    @pl.when(e >= NUM_ROUTED_E)
    def _identity():
        ys_ref[...] = ws1 * xs_ref[...]


def _ffn_call(bexp, xs, ws, gate_up_proj, down_proj):
    def _blk(b, be):
        return jnp.where(be[b] > NUM_ROUTED_E, NBLK - 1, b)

    def _exp(b, be):
        return jnp.minimum(be[b], 7)

    return pl.pallas_call(
        _ffn_body,
        grid_spec=pltpu.PrefetchScalarGridSpec(
            num_scalar_prefetch=1,
            grid=(NBLK,),
            in_specs=[
                pl.BlockSpec((BLK, HIDDEN_D), lambda b, be: (_blk(b, be), 0)),
                pl.BlockSpec((BLK, WSW), lambda b, be: (_blk(b, be), 0)),
                pl.BlockSpec((1, FFN_D, HIDDEN_D),
                             lambda b, be: (_exp(b, be), 0, 0)),
                pl.BlockSpec((1, FFN_D, HIDDEN_D),
                             lambda b, be: (_exp(b, be), 1, 0)),
                pl.BlockSpec((1, HIDDEN_D, FFN_D),
                             lambda b, be: (_exp(b, be), 0, 0)),
            ],
            out_specs=pl.BlockSpec((BLK, HIDDEN_D),
                                   lambda b, be: (_blk(b, be), 0)),
        ),
        out_shape=jax.ShapeDtypeStruct((NP, HIDDEN_D), jnp.float32),
        compiler_params=pltpu.CompilerParams(
            dimension_semantics=("arbitrary",),
        ),
    )(bexp, xs, ws, gate_up_proj, gate_up_proj, down_proj)


def kernel(hidden_states, top_k_index, top_k_weights, gate_up_proj, down_proj):
    T, H = hidden_states.shape
    e2 = top_k_index.reshape(ROWS_R, COLS_C)

    pos2, bexp2 = pl.pallas_call(
        _route_body,
        out_shape=(
            jax.ShapeDtypeStruct((ROWS_R, COLS_C), jnp.int32),
            jax.ShapeDtypeStruct((1, COLS_C), jnp.int32),
        ),
    )(e2)
    pos = pos2.reshape(T)
    bexp = bexp2.reshape(COLS_C)

    scmesh = plsc.VectorSubcoreMesh(core_axis_name="c", subcore_axis_name="s")

    wt16 = jnp.tile(top_k_weights, (1, WSW))

    xs, ws = pl.kernel(
        _sc_scatter_body,
        out_type=(
            jax.ShapeDtypeStruct((NP, H), jnp.float32),
            jax.ShapeDtypeStruct((NP, WSW), jnp.float32),
        ),
        mesh=scmesh,
        scratch_types=[
            pltpu.VMEM((HALF,), jnp.int32),
            pltpu.VMEM((HALF,), jnp.int32),
            pltpu.VMEM((HALF, H), jnp.float32),
            pltpu.VMEM((HALF, H), jnp.float32),
            pltpu.VMEM((HALF, WSW), jnp.float32),
            pltpu.VMEM((HALF, WSW), jnp.float32),
            pltpu.SemaphoreType.DMA,
            pltpu.SemaphoreType.DMA,
            pltpu.SemaphoreType.DMA,
        ],
    )(hidden_states, wt16, pos)

    ys = _ffn_call(bexp, xs, ws, gate_up_proj, down_proj)

    out = pl.kernel(
        _sc_gather_body,
        out_type=jax.ShapeDtypeStruct((T, H), jnp.float32),
        mesh=scmesh,
        scratch_types=[
            pltpu.VMEM((HALF,), jnp.int32),
            pltpu.VMEM((HALF,), jnp.int32),
            pltpu.VMEM((HALF, H), jnp.float32),
            pltpu.VMEM((HALF, H), jnp.float32),
            pltpu.SemaphoreType.DMA,
            pltpu.SemaphoreType.DMA,
        ],
    )(ys, pos)
    return out
